# Initial kernel scaffold; baseline (speedup 1.0000x reference)
import jax
import jax.numpy as jnp
from jax.experimental import pallas as pl


def kernel(indices, anchor_hashes, node_hashes, relations, anchor_emb, W1, b1, W2, b2):
    raise NotImplementedError("write your pallas kernel here")


# speedup vs baseline: 4.8831x; 4.8831x over previous
import functools

import jax
import jax.numpy as jnp
from jax import lax
from jax.experimental import pallas as pl
from jax.experimental.pallas import tpu as pltpu
from jax.experimental.pallas import tpu_sc as plsc

NUM_NODES = 100000
ANCS = 20
RELCTX = 12
NTOK = ANCS + RELCTX
DIM = 64
B = 16384

_info = plsc.get_sparse_core_info()
NC, NS = _info.num_cores, _info.num_subcores
NW = NC * NS

NODES_PER_W = B // NW
ANC_TOK_PER_W = NODES_PER_W * ANCS
REL_TOK_PER_W = NODES_PER_W * RELCTX
CHUNK = 256
ANC_CHUNKS = ANC_TOK_PER_W // CHUNK
REL_CHUNKS = REL_TOK_PER_W // CHUNK

_mesh = plsc.VectorSubcoreMesh(core_axis_name="c", subcore_axis_name="s")
_sc_params = pltpu.CompilerParams(use_tc_tiling_on_sc=False)


@functools.partial(
    pl.kernel,
    mesh=_mesh,
    compiler_params=_sc_params,
    out_type=jax.ShapeDtypeStruct((B, NTOK), jnp.int32),
    scratch_types=[
        pltpu.VMEM((NODES_PER_W,), jnp.int32),
        pltpu.VMEM((NODES_PER_W, NTOK), jnp.int32),
        pltpu.SemaphoreType.DMA,
    ],
)
def _gather_tokens(idx_hbm, hashes_hbm, tok_hbm, idx_v, tok_v, sem):
    wid = lax.axis_index("s") * NC + lax.axis_index("c")
    base = wid * NODES_PER_W
    pltpu.sync_copy(idx_hbm.at[pl.ds(base, NODES_PER_W)], idx_v)
    pltpu.async_copy(hashes_hbm.at[idx_v], tok_v, sem).wait()
    pltpu.sync_copy(tok_v, tok_hbm.at[pl.ds(base, NODES_PER_W)])


@functools.partial(
    pl.kernel,
    mesh=_mesh,
    compiler_params=_sc_params,
    out_type=[
        jax.ShapeDtypeStruct((NW, ANC_CHUNKS, CHUNK, DIM), jnp.float32),
        jax.ShapeDtypeStruct((NW, REL_CHUNKS, CHUNK, DIM), jnp.float32),
    ],
    scratch_types=[
        pltpu.VMEM((ANC_CHUNKS, CHUNK), jnp.int32),
        pltpu.VMEM((REL_CHUNKS, CHUNK), jnp.int32),
        pltpu.VMEM((2, CHUNK, DIM), jnp.float32),
        pltpu.SemaphoreType.DMA((2,)),
    ],
)
def _gather_embs(anc_emb_hbm, rel_emb_hbm, anc_tok_hbm, rel_tok_hbm,
                 anc_out_hbm, rel_out_hbm, ia_v, ir_v, buf, sem):
    wid = lax.axis_index("s") * NC + lax.axis_index("c")
    pltpu.sync_copy(anc_tok_hbm.at[wid], ia_v)
    pltpu.sync_copy(rel_tok_hbm.at[wid], ir_v)

    def run(table_hbm, idx_v, out_hbm, nchunks):
        pltpu.async_copy(table_hbm.at[idx_v.at[0]], buf.at[0], sem.at[0])

        def body(c, _):
            nxt = c + 1

            @pl.when(nxt < nchunks)
            def _():
                pltpu.make_async_copy(
                    table_hbm.at[idx_v.at[nxt]], buf.at[nxt % 2], sem.at[nxt % 2]
                ).start()

            pltpu.make_async_copy(
                table_hbm.at[idx_v.at[c]], buf.at[c % 2], sem.at[c % 2]
            ).wait()
            pltpu.sync_copy(buf.at[c % 2], out_hbm.at[wid, c])
            return 0

        lax.fori_loop(0, nchunks, body, 0)

    run(anc_emb_hbm, ia_v, anc_out_hbm, ANC_CHUNKS)
    run(rel_emb_hbm, ir_v, rel_out_hbm, REL_CHUNKS)


ROWS_BLK = 512


def _mlp_body(anc_ref, rel_ref, w1a_ref, w1b_ref, b1_ref, w2_ref, b2_ref,
              out_ref):
    h = jnp.dot(anc_ref[...], w1a_ref[...], preferred_element_type=jnp.float32)
    h = h + jnp.dot(rel_ref[...], w1b_ref[...],
                    preferred_element_type=jnp.float32)
    h = jnp.maximum(h + b1_ref[...], 0.0)
    out_ref[...] = (
        jnp.dot(h, w2_ref[...], preferred_element_type=jnp.float32)
        + b2_ref[...]
    )


def _mlp(anc_flat, rel_flat, w1a, w1b, b1, w2, b2):
    grid = (B // ROWS_BLK,)
    return pl.pallas_call(
        _mlp_body,
        grid=grid,
        in_specs=[
            pl.BlockSpec((ROWS_BLK, ANCS * DIM), lambda i: (i, 0)),
            pl.BlockSpec((ROWS_BLK, RELCTX * DIM), lambda i: (i, 0)),
            pl.BlockSpec((ANCS * DIM, 2 * DIM), lambda i: (0, 0)),
            pl.BlockSpec((RELCTX * DIM, 2 * DIM), lambda i: (0, 0)),
            pl.BlockSpec((1, 2 * DIM), lambda i: (0, 0)),
            pl.BlockSpec((2 * DIM, DIM), lambda i: (0, 0)),
            pl.BlockSpec((1, DIM), lambda i: (0, 0)),
        ],
        out_specs=pl.BlockSpec((ROWS_BLK, DIM), lambda i: (i, 0)),
        out_shape=jax.ShapeDtypeStruct((B, DIM), jnp.float32),
    )(anc_flat, rel_flat, w1a, w1b, b1, w2, b2)


def kernel(indices, anchor_hashes, node_hashes, relations, anchor_emb,
           W1, b1, W2, b2):
    hashes = jnp.concatenate([anchor_hashes, node_hashes], axis=1)
    tok = _gather_tokens(indices, hashes)
    anc_tok = tok[:, :ANCS].reshape(NW, ANC_CHUNKS, CHUNK)
    rel_tok = tok[:, ANCS:].reshape(NW, REL_CHUNKS, CHUNK)
    anc_rows, rel_rows = _gather_embs(anchor_emb, relations, anc_tok, rel_tok)
    anc_flat = anc_rows.reshape(B, ANCS * DIM)
    rel_flat = rel_rows.reshape(B, RELCTX * DIM)
    return _mlp(anc_flat, rel_flat, W1[: ANCS * DIM], W1[ANCS * DIM:],
                b1.reshape(1, -1), W2, b2.reshape(1, -1))
